# Initial kernel scaffold; baseline (speedup 1.0000x reference)
import jax
import jax.numpy as jnp
from jax.experimental import pallas as pl


def kernel(edge_index, nodes, Wg, bg, bn1_g, bn1_b, W1, b1, W2, bn2_g, bn2_b, cls_W, cls_b):
    raise NotImplementedError("write your pallas kernel here")


# speedup vs baseline: 41.4825x; 41.4825x over previous
import functools

import jax
import jax.numpy as jnp
from jax import lax
from jax.experimental import pallas as pl
from jax.experimental.pallas import tpu as pltpu
from jax.experimental.pallas import tpu_sc as plsc

N = 100000
E = 3200000
EMB = 16
NUMCLS = 40

NC = 2
NS = 16
NW = NC * NS

NP = 100096
NPAD = NP - N
SLAB = NP // NS
EPT = E // NW
SUB = 80
ROWS0 = EPT // SUB
PADR = 6
ROWS = ROWS0 + PADR
NSUB = 8
NGRP = ROWS // NSUB
ZR = 272

NB = N // 8
BR = 1250
GSTEPS = NB // BR

_f32 = jnp.float32


def _mesh():
    return plsc.VectorSubcoreMesh(core_axis_name="c", subcore_axis_name="s")


_SC_PARAMS = pltpu.CompilerParams(use_tc_tiling_on_sc=False)


def _deg_body(dst2d, out_hbm, idx_v, ones_v, zbuf_v, deg_sp):
    c = lax.axis_index("c")
    s = lax.axis_index("s")
    w = c * NS + s

    def fill_ones(i, _):
        ones_v[pl.ds(i * 16, 16)] = jnp.ones((16,), _f32)
        return 0

    lax.fori_loop(0, SUB // 16, fill_ones, 0)

    def fill_zero(i, _):
        zbuf_v[pl.ds(i * 16, 16)] = jnp.zeros((16,), _f32)
        return 0

    lax.fori_loop(0, SLAB // 16, fill_zero, 0)
    pltpu.sync_copy(zbuf_v, deg_sp.at[pl.ds(s * SLAB, SLAB)])
    plsc.subcore_barrier()

    def grp(g, _):
        pltpu.sync_copy(dst2d.at[w, pl.ds(g * NSUB, NSUB), :], idx_v)
        for j in range(NSUB):
            pltpu.sync_copy(ones_v, deg_sp.at[idx_v.at[j]], add=True)
        return 0

    lax.fori_loop(0, NGRP, grp, 0)
    plsc.subcore_barrier()
    pltpu.sync_copy(deg_sp.at[pl.ds(s * SLAB, SLAB)], zbuf_v)
    pltpu.sync_copy(zbuf_v, out_hbm.at[pl.ds(c * NP + s * SLAB, SLAB)])


def _deg_call(dst2d):
    f = pl.kernel(
        _deg_body,
        out_type=jax.ShapeDtypeStruct((NC * NP,), _f32),
        mesh=_mesh(),
        scratch_types=[
            pltpu.VMEM((NSUB, SUB), jnp.int32),
            pltpu.VMEM((SUB,), _f32),
            pltpu.VMEM((SLAB,), _f32),
            pltpu.VMEM_SHARED((NP,), _f32),
        ],
        compiler_params=_SC_PARAMS,
    )
    return f(dst2d)


def _agg_body(xs_hbm, src2d, dst2d, out_hbm,
              idxs_v, idxd_v, rows_v, zrow_v, ys_sp, gsem):
    c = lax.axis_index("c")
    s = lax.axis_index("s")
    w = c * NS + s

    def fill_zero(i, _):
        zrow_v[i] = jnp.zeros((16,), _f32)
        return 0

    lax.fori_loop(0, ZR, fill_zero, 0)
    for k in range(SLAB // ZR):
        pltpu.sync_copy(zrow_v, ys_sp.at[pl.ds(s * SLAB + k * ZR, ZR), :])
    plsc.subcore_barrier()

    def grp(g, _):
        pltpu.sync_copy(src2d.at[w, pl.ds(g * NSUB, NSUB), :], idxs_v)
        pltpu.sync_copy(dst2d.at[w, pl.ds(g * NSUB, NSUB), :], idxd_v)
        descs = [
            pltpu.async_copy(xs_hbm.at[idxs_v.at[j]],
                             rows_v.at[pl.ds(j * SUB, SUB), :], gsem)
            for j in range(NSUB)
        ]
        for d in descs:
            d.wait()
        for j in range(NSUB):
            pltpu.sync_copy(rows_v.at[pl.ds(j * SUB, SUB), :],
                            ys_sp.at[idxd_v.at[j]], add=True)
        return 0

    lax.fori_loop(0, NGRP, grp, 0)
    plsc.subcore_barrier()
    for k in range(SLAB // ZR):
        r0 = s * SLAB + k * ZR
        pltpu.sync_copy(ys_sp.at[pl.ds(r0, ZR), :], zrow_v)
        pltpu.sync_copy(zrow_v, out_hbm.at[c, pl.ds(r0, ZR), :])


def _agg_call(xs, src2d, dst2d):
    f = pl.kernel(
        _agg_body,
        out_type=jax.ShapeDtypeStruct((NC, NP, EMB), _f32),
        mesh=_mesh(),
        scratch_types=[
            pltpu.VMEM((NSUB, SUB), jnp.int32),
            pltpu.VMEM((NSUB, SUB), jnp.int32),
            pltpu.VMEM((NSUB * SUB, EMB), _f32),
            pltpu.VMEM((ZR, EMB), _f32),
            pltpu.VMEM_SHARED((NP, EMB), _f32),
            pltpu.SemaphoreType.DMA,
        ],
        compiler_params=_SC_PARAMS,
    )
    return f(xs, src2d, dst2d)


def _row_spec():
    return pl.BlockSpec((1, BR, 128), lambda j: (j, 0, 0))


def _prep_body(degp_ref, x_ref, r8_ref, xs_ref, rdeg_ref, inv_ref):
    d = degp_ref[0, 0] + degp_ref[1, 0] + 1.0
    dt = jnp.dot(d, r8_ref[...], preferred_element_type=_f32)
    rdeg = lax.rsqrt(dt)
    rdeg_ref[0] = rdeg
    inv_ref[0] = 1.0 / dt
    xs_ref[0] = x_ref[0] * rdeg


def _prep_call(degp_t, x_t, r8):
    return pl.pallas_call(
        _prep_body,
        grid=(GSTEPS,),
        in_specs=[
            pl.BlockSpec((2, 1, BR, 8), lambda j: (0, j, 0, 0)),
            _row_spec(),
            pl.BlockSpec((8, 128), lambda j: (0, 0)),
        ],
        out_specs=[_row_spec(), _row_spec(), _row_spec()],
        out_shape=[jax.ShapeDtypeStruct((GSTEPS, BR, 128), _f32)] * 3,
    )(degp_t, x_t, r8)


def _d1_body(ysp_ref, x_ref, rdeg_ref, inv_ref, w_ref, b_ref,
             x1_ref, st_ref, acc):
    j = pl.program_id(0)
    x = x_ref[0]
    agg = rdeg_ref[0] * (ysp_ref[0, 0] + ysp_ref[1, 0]) + x * inv_ref[0]
    h = jnp.maximum(
        jnp.dot(agg, w_ref[...], preferred_element_type=_f32) + b_ref[0:1],
        0.0)
    x1 = h + x
    x1_ref[0] = x1

    @pl.when(j == 0)
    def _():
        acc[...] = jnp.zeros_like(acc)

    ssum = jnp.sum(x1, axis=0, keepdims=True)
    ssq = jnp.sum(x1 * x1, axis=0, keepdims=True)
    acc[...] += jnp.concatenate([ssum, ssq], axis=0)

    @pl.when(j == GSTEPS - 1)
    def _():
        st_ref[...] = acc[...]


def _d1_call(ysp_t, x_t, rdeg_t, inv_t, wg_t, bg_t):
    return pl.pallas_call(
        _d1_body,
        grid=(GSTEPS,),
        in_specs=[
            pl.BlockSpec((2, 1, BR, 128), lambda j: (0, j, 0, 0)),
            _row_spec(), _row_spec(), _row_spec(),
            pl.BlockSpec((128, 128), lambda j: (0, 0)),
            pl.BlockSpec((8, 128), lambda j: (0, 0)),
        ],
        out_specs=[_row_spec(), pl.BlockSpec((2, 128), lambda j: (0, 0))],
        out_shape=[jax.ShapeDtypeStruct((GSTEPS, BR, 128), _f32),
                   jax.ShapeDtypeStruct((2, 128), _f32)],
        scratch_shapes=[pltpu.VMEM((2, 128), _f32)],
    )(ysp_t, x_t, rdeg_t, inv_t, wg_t, bg_t)


def _bn_affine(st, g_ref, b_ref, t_ref, tt_ref):
    s16 = jnp.dot(st, tt_ref[...], preferred_element_type=_f32)
    mean = s16[0:1] / N
    var = s16[1:2] / N - mean * mean
    istd = lax.rsqrt(var + 1e-5)
    scale = istd * g_ref[...]
    shift = b_ref[...] - mean * scale
    scale_t = jnp.dot(scale, t_ref[...], preferred_element_type=_f32)
    shift_t = jnp.dot(shift, t_ref[...], preferred_element_type=_f32)
    return scale_t, shift_t


def _d2_body(x1_ref, st_ref, g_ref, b_ref, t_ref, tt_ref,
             w1_ref, b1_ref, w2_ref, x2_ref, st2_ref, acc):
    j = pl.program_id(0)
    scale_t, shift_t = _bn_affine(st_ref[...], g_ref, b_ref, t_ref, tt_ref)
    x1n = x1_ref[0] * scale_t + shift_t
    h = jnp.maximum(
        jnp.dot(x1n, w1_ref[...], preferred_element_type=_f32) + b1_ref[0:1],
        0.0)
    x2 = jnp.dot(h, w2_ref[...], preferred_element_type=_f32) + x1n
    x2_ref[0] = x2

    @pl.when(j == 0)
    def _():
        acc[...] = jnp.zeros_like(acc)

    ssum = jnp.sum(x2, axis=0, keepdims=True)
    ssq = jnp.sum(x2 * x2, axis=0, keepdims=True)
    acc[...] += jnp.concatenate([ssum, ssq], axis=0)

    @pl.when(j == GSTEPS - 1)
    def _():
        st2_ref[...] = acc[...]


def _d2_call(x1_t, st1, g16, b16, t16, tt16, w1_t, b1_t, w2_t):
    return pl.pallas_call(
        _d2_body,
        grid=(GSTEPS,),
        in_specs=[
            _row_spec(),
            pl.BlockSpec((2, 128), lambda j: (0, 0)),
            pl.BlockSpec((1, 16), lambda j: (0, 0)),
            pl.BlockSpec((1, 16), lambda j: (0, 0)),
            pl.BlockSpec((16, 128), lambda j: (0, 0)),
            pl.BlockSpec((128, 16), lambda j: (0, 0)),
            pl.BlockSpec((128, 512), lambda j: (0, 0)),
            pl.BlockSpec((8, 512), lambda j: (0, 0)),
            pl.BlockSpec((512, 128), lambda j: (0, 0)),
        ],
        out_specs=[_row_spec(), pl.BlockSpec((2, 128), lambda j: (0, 0))],
        out_shape=[jax.ShapeDtypeStruct((GSTEPS, BR, 128), _f32),
                   jax.ShapeDtypeStruct((2, 128), _f32)],
        scratch_shapes=[pltpu.VMEM((2, 128), _f32)],
    )(x1_t, st1, g16, b16, t16, tt16, w1_t, b1_t, w2_t)


def _d3a_body(x2_ref, st_ref, g_ref, b_ref, t_ref, tt_ref, rdeg_ref,
              xo_ref, xso_ref):
    scale_t, shift_t = _bn_affine(st_ref[...], g_ref, b_ref, t_ref, tt_ref)
    xn = x2_ref[0] * scale_t + shift_t
    xo_ref[0] = xn
    xso_ref[0] = xn * rdeg_ref[0]


def _d3a_call(x2_t, st2, g16, b16, t16, tt16, rdeg_t):
    return pl.pallas_call(
        _d3a_body,
        grid=(GSTEPS,),
        in_specs=[
            _row_spec(),
            pl.BlockSpec((2, 128), lambda j: (0, 0)),
            pl.BlockSpec((1, 16), lambda j: (0, 0)),
            pl.BlockSpec((1, 16), lambda j: (0, 0)),
            pl.BlockSpec((16, 128), lambda j: (0, 0)),
            pl.BlockSpec((128, 16), lambda j: (0, 0)),
            _row_spec(),
        ],
        out_specs=[_row_spec(), _row_spec()],
        out_shape=[jax.ShapeDtypeStruct((GSTEPS, BR, 128), _f32)] * 2,
    )(x2_t, st2, g16, b16, t16, tt16, rdeg_t)


def _d3b_body(x2_ref, st_ref, g_ref, b_ref, t_ref, tt_ref, wc_ref, bc_ref,
              y_ref):
    scale_t, shift_t = _bn_affine(st_ref[...], g_ref, b_ref, t_ref, tt_ref)
    xn = x2_ref[0] * scale_t + shift_t
    y_ref[0] = (jnp.dot(xn, wc_ref[...], preferred_element_type=_f32)
                + bc_ref[0:1])


def _d3b_call(x2_t, st2, g16, b16, t16, tt16, wc_t, bc_t):
    return pl.pallas_call(
        _d3b_body,
        grid=(GSTEPS,),
        in_specs=[
            _row_spec(),
            pl.BlockSpec((2, 128), lambda j: (0, 0)),
            pl.BlockSpec((1, 16), lambda j: (0, 0)),
            pl.BlockSpec((1, 16), lambda j: (0, 0)),
            pl.BlockSpec((16, 128), lambda j: (0, 0)),
            pl.BlockSpec((128, 16), lambda j: (0, 0)),
            pl.BlockSpec((128, 320), lambda j: (0, 0)),
            pl.BlockSpec((8, 320), lambda j: (0, 0)),
        ],
        out_specs=pl.BlockSpec((1, BR, 320), lambda j: (j, 0, 0)),
        out_shape=jax.ShapeDtypeStruct((GSTEPS, BR, 320), _f32),
    )(x2_t, st2, g16, b16, t16, tt16, wc_t, bc_t)


def _kron8(w):
    return jnp.kron(jnp.eye(8, dtype=_f32), w)


def _tile_bias(b, reps, rows=8):
    return jnp.broadcast_to(jnp.tile(b, reps)[None, :], (rows, b.shape[0] * reps))


def kernel(edge_index, nodes, Wg, bg, bn1_g, bn1_b, W1, b1, W2, bn2_g, bn2_b,
           cls_W, cls_b):
    padv = (N + (jnp.arange(PADR * SUB, dtype=jnp.int32) % NPAD))
    padv = jnp.broadcast_to(padv.reshape(1, PADR, SUB), (NW, PADR, SUB))

    def _edges3d(e):
        return jnp.concatenate([e.reshape(NW, ROWS0, SUB), padv], axis=1)

    src2d = _edges3d(edge_index[0])
    dst2d = _edges3d(edge_index[1])

    r8 = jnp.repeat(jnp.eye(8, dtype=_f32), 16, axis=1)
    t16 = jnp.tile(jnp.eye(16, dtype=_f32), (1, 8))
    tt16 = t16.T

    degp = _deg_call(dst2d).reshape(NC, NP)
    degp_t = degp[:, :N].reshape(2, GSTEPS, BR, 8)
    x_t = nodes.reshape(GSTEPS, BR, 128)
    xs_t, rdeg_t, inv_t = _prep_call(degp_t, x_t, r8)

    y_t = None
    for i in range(2):
        xs = jnp.concatenate(
            [xs_t.reshape(N, EMB), jnp.zeros((NPAD, EMB), _f32)], axis=0)
        ysp = _agg_call(xs, src2d, dst2d)
        ysp_t = ysp[:, :N, :].reshape(2, GSTEPS, BR, 128)
        x1_t, st1 = _d1_call(ysp_t, x_t, rdeg_t, inv_t,
                             _kron8(Wg[i]), _tile_bias(bg[i], 8))
        x2_t, st2 = _d2_call(x1_t, st1,
                             bn1_g[i][None, :], bn1_b[i][None, :], t16, tt16,
                             _kron8(W1[i]), _tile_bias(b1[i], 8),
                             _kron8(W2[i]))
        if i == 0:
            x_t, xs_t = _d3a_call(x2_t, st2, bn2_g[i][None, :],
                                  bn2_b[i][None, :], t16, tt16, rdeg_t)
        else:
            y_t = _d3b_call(x2_t, st2, bn2_g[i][None, :], bn2_b[i][None, :],
                            t16, tt16, _kron8(cls_W),
                            _tile_bias(cls_b, 8))
    return y_t.reshape(N, NUMCLS)


# speedup vs baseline: 64.0079x; 1.5430x over previous
import functools

import jax
import jax.numpy as jnp
from jax import lax
from jax.experimental import pallas as pl
from jax.experimental.pallas import tpu as pltpu
from jax.experimental.pallas import tpu_sc as plsc

N = 100000
E = 3200000
EMB = 16
NUMCLS = 40

NC = 2
NS = 16
NW = NC * NS

SLAB = N // NS
SLAB_A = 6256
SLAB_L = N - (NS - 1) * SLAB_A
EPT = E // NW
SUB = 80
ROWS = EPT // SUB
NSUB = 8
NGRP = ROWS // NSUB
TAIL = ROWS - NGRP * NSUB
ZR = 250

NB = N // 8
BR = 1250
GSTEPS = NB // BR

_f32 = jnp.float32


def _mesh():
    return plsc.VectorSubcoreMesh(core_axis_name="c", subcore_axis_name="s")


_SC_PARAMS = pltpu.CompilerParams(use_tc_tiling_on_sc=False)


def _deg_body(er, out_hbm, idx_v, ones_v, zbuf_v, deg_sp):
    c = lax.axis_index("c")
    s = lax.axis_index("s")
    w = c * NS + s

    def fill_ones(i, _):
        ones_v[pl.ds(i * 16, 16)] = jnp.ones((16,), _f32)
        return 0

    lax.fori_loop(0, SUB // 16, fill_ones, 0)

    def fill_zero(i, _):
        zbuf_v[pl.ds(i * 16, 16)] = jnp.zeros((16,), _f32)
        return 0

    lax.fori_loop(0, SLAB_A // 16, fill_zero, 0)

    @pl.when(s < NS - 1)
    def _():
        pltpu.sync_copy(zbuf_v, deg_sp.at[pl.ds(s * SLAB_A, SLAB_A)])

    @pl.when(s == NS - 1)
    def _():
        pltpu.sync_copy(zbuf_v.at[pl.ds(0, SLAB_L)],
                        deg_sp.at[pl.ds(s * SLAB_A, SLAB_L)])

    plsc.subcore_barrier()

    def grp(g, _):
        pltpu.sync_copy(er.at[1, w, pl.ds(g * NSUB, NSUB), :], idx_v)
        for j in range(NSUB):
            pltpu.sync_copy(ones_v, deg_sp.at[idx_v.at[j]], add=True)
        return 0

    lax.fori_loop(0, NGRP, grp, 0)
    pltpu.sync_copy(er.at[1, w, pl.ds(NGRP * NSUB, TAIL), :],
                    idx_v.at[pl.ds(0, TAIL), :])
    for j in range(TAIL):
        pltpu.sync_copy(ones_v, deg_sp.at[idx_v.at[j]], add=True)
    plsc.subcore_barrier()

    @pl.when(s < NS - 1)
    def _():
        pltpu.sync_copy(deg_sp.at[pl.ds(s * SLAB_A, SLAB_A)], zbuf_v)
        pltpu.sync_copy(zbuf_v, out_hbm.at[pl.ds(c * N + s * SLAB_A, SLAB_A)])

    @pl.when(s == NS - 1)
    def _():
        pltpu.sync_copy(deg_sp.at[pl.ds(s * SLAB_A, SLAB_L)],
                        zbuf_v.at[pl.ds(0, SLAB_L)])
        pltpu.sync_copy(zbuf_v.at[pl.ds(0, SLAB_L)],
                        out_hbm.at[pl.ds(c * N + s * SLAB_A, SLAB_L)])


def _deg_call(er):
    f = pl.kernel(
        _deg_body,
        out_type=jax.ShapeDtypeStruct((NC * N,), _f32),
        mesh=_mesh(),
        scratch_types=[
            pltpu.VMEM((NSUB, SUB), jnp.int32),
            pltpu.VMEM((SUB,), _f32),
            pltpu.VMEM((SLAB_A,), _f32),
            pltpu.VMEM_SHARED((N,), _f32),
        ],
        compiler_params=_SC_PARAMS,
    )
    return f(er)


def _agg_body(xs_hbm, er, out_hbm,
              idxs_v, idxd_v, rows_v, zrow_v, ys_sp, gsem):
    c = lax.axis_index("c")
    s = lax.axis_index("s")
    w = c * NS + s

    def fill_zero(i, _):
        zrow_v[i] = jnp.zeros((16,), _f32)
        return 0

    lax.fori_loop(0, ZR, fill_zero, 0)
    for k in range(SLAB // ZR):
        pltpu.sync_copy(zrow_v, ys_sp.at[pl.ds(s * SLAB + k * ZR, ZR), :])
    plsc.subcore_barrier()

    def chunk(nsub):
        def go(g, _):
            pltpu.sync_copy(er.at[0, w, pl.ds(g * NSUB, nsub), :],
                            idxs_v.at[pl.ds(0, nsub), :])
            pltpu.sync_copy(er.at[1, w, pl.ds(g * NSUB, nsub), :],
                            idxd_v.at[pl.ds(0, nsub), :])
            descs = [
                pltpu.async_copy(xs_hbm.at[idxs_v.at[j]],
                                 rows_v.at[pl.ds(j * SUB, SUB), :], gsem)
                for j in range(nsub)
            ]
            for d in descs:
                d.wait()
            for j in range(nsub):
                pltpu.sync_copy(rows_v.at[pl.ds(j * SUB, SUB), :],
                                ys_sp.at[idxd_v.at[j]], add=True)
            return 0
        return go

    lax.fori_loop(0, NGRP, chunk(NSUB), 0)
    chunk(TAIL)(NGRP, 0)
    plsc.subcore_barrier()
    for k in range(SLAB // ZR):
        r0 = s * SLAB + k * ZR
        pltpu.sync_copy(ys_sp.at[pl.ds(r0, ZR), :], zrow_v)
        pltpu.sync_copy(zrow_v, out_hbm.at[c, pl.ds(r0, ZR), :])


def _agg_call(xs, er):
    f = pl.kernel(
        _agg_body,
        out_type=jax.ShapeDtypeStruct((NC, N, EMB), _f32),
        mesh=_mesh(),
        scratch_types=[
            pltpu.VMEM((NSUB, SUB), jnp.int32),
            pltpu.VMEM((NSUB, SUB), jnp.int32),
            pltpu.VMEM((NSUB * SUB, EMB), _f32),
            pltpu.VMEM((ZR, EMB), _f32),
            pltpu.VMEM_SHARED((N, EMB), _f32),
            pltpu.SemaphoreType.DMA,
        ],
        compiler_params=_SC_PARAMS,
    )
    return f(xs, er)


def _row_spec():
    return pl.BlockSpec((1, BR, 128), lambda j: (j, 0, 0))


def _prep_body(degp_ref, x_ref, r8_ref, xs_ref, rdeg_ref, inv_ref):
    d = degp_ref[0, 0] + degp_ref[1, 0] + 1.0
    dt = jnp.dot(d, r8_ref[...], preferred_element_type=_f32)
    rdeg = lax.rsqrt(dt)
    rdeg_ref[0] = rdeg
    inv_ref[0] = 1.0 / dt
    xs_ref[0] = x_ref[0] * rdeg


def _prep_call(degp_t, x_t, r8):
    return pl.pallas_call(
        _prep_body,
        grid=(GSTEPS,),
        in_specs=[
            pl.BlockSpec((2, 1, BR, 8), lambda j: (0, j, 0, 0)),
            _row_spec(),
            pl.BlockSpec((8, 128), lambda j: (0, 0)),
        ],
        out_specs=[_row_spec(), _row_spec(), _row_spec()],
        out_shape=[jax.ShapeDtypeStruct((GSTEPS, BR, 128), _f32)] * 3,
    )(degp_t, x_t, r8)


def _d1_body(ysp_ref, x_ref, rdeg_ref, inv_ref, w_ref, b_ref,
             x1_ref, st_ref, acc):
    j = pl.program_id(0)
    x = x_ref[0]
    agg = rdeg_ref[0] * (ysp_ref[0, 0] + ysp_ref[1, 0]) + x * inv_ref[0]
    h = jnp.maximum(
        jnp.dot(agg, w_ref[...], preferred_element_type=_f32) + b_ref[0:1],
        0.0)
    x1 = h + x
    x1_ref[0] = x1

    @pl.when(j == 0)
    def _():
        acc[...] = jnp.zeros_like(acc)

    ssum = jnp.sum(x1, axis=0, keepdims=True)
    ssq = jnp.sum(x1 * x1, axis=0, keepdims=True)
    acc[...] += jnp.concatenate([ssum, ssq], axis=0)

    @pl.when(j == GSTEPS - 1)
    def _():
        st_ref[...] = acc[...]


def _d1_call(ysp_t, x_t, rdeg_t, inv_t, wg_t, bg_t):
    return pl.pallas_call(
        _d1_body,
        grid=(GSTEPS,),
        in_specs=[
            pl.BlockSpec((2, 1, BR, 128), lambda j: (0, j, 0, 0)),
            _row_spec(), _row_spec(), _row_spec(),
            pl.BlockSpec((128, 128), lambda j: (0, 0)),
            pl.BlockSpec((8, 128), lambda j: (0, 0)),
        ],
        out_specs=[_row_spec(), pl.BlockSpec((2, 128), lambda j: (0, 0))],
        out_shape=[jax.ShapeDtypeStruct((GSTEPS, BR, 128), _f32),
                   jax.ShapeDtypeStruct((2, 128), _f32)],
        scratch_shapes=[pltpu.VMEM((2, 128), _f32)],
    )(ysp_t, x_t, rdeg_t, inv_t, wg_t, bg_t)


def _bn_affine(st, g_ref, b_ref, t_ref, tt_ref):
    s16 = jnp.dot(st, tt_ref[...], preferred_element_type=_f32)
    mean = s16[0:1] / N
    var = s16[1:2] / N - mean * mean
    istd = lax.rsqrt(var + 1e-5)
    scale = istd * g_ref[...]
    shift = b_ref[...] - mean * scale
    scale_t = jnp.dot(scale, t_ref[...], preferred_element_type=_f32)
    shift_t = jnp.dot(shift, t_ref[...], preferred_element_type=_f32)
    return scale_t, shift_t


def _d2_body(x1_ref, st_ref, g_ref, b_ref, t_ref, tt_ref,
             w1_ref, b1_ref, w2_ref, x2_ref, st2_ref, acc):
    j = pl.program_id(0)
    scale_t, shift_t = _bn_affine(st_ref[...], g_ref, b_ref, t_ref, tt_ref)
    x1n = x1_ref[0] * scale_t + shift_t
    h = jnp.maximum(
        jnp.dot(x1n, w1_ref[...], preferred_element_type=_f32) + b1_ref[0:1],
        0.0)
    x2 = jnp.dot(h, w2_ref[...], preferred_element_type=_f32) + x1n
    x2_ref[0] = x2

    @pl.when(j == 0)
    def _():
        acc[...] = jnp.zeros_like(acc)

    ssum = jnp.sum(x2, axis=0, keepdims=True)
    ssq = jnp.sum(x2 * x2, axis=0, keepdims=True)
    acc[...] += jnp.concatenate([ssum, ssq], axis=0)

    @pl.when(j == GSTEPS - 1)
    def _():
        st2_ref[...] = acc[...]


def _d2_call(x1_t, st1, g16, b16, t16, tt16, w1_t, b1_t, w2_t):
    return pl.pallas_call(
        _d2_body,
        grid=(GSTEPS,),
        in_specs=[
            _row_spec(),
            pl.BlockSpec((2, 128), lambda j: (0, 0)),
            pl.BlockSpec((1, 16), lambda j: (0, 0)),
            pl.BlockSpec((1, 16), lambda j: (0, 0)),
            pl.BlockSpec((16, 128), lambda j: (0, 0)),
            pl.BlockSpec((128, 16), lambda j: (0, 0)),
            pl.BlockSpec((128, 512), lambda j: (0, 0)),
            pl.BlockSpec((8, 512), lambda j: (0, 0)),
            pl.BlockSpec((512, 128), lambda j: (0, 0)),
        ],
        out_specs=[_row_spec(), pl.BlockSpec((2, 128), lambda j: (0, 0))],
        out_shape=[jax.ShapeDtypeStruct((GSTEPS, BR, 128), _f32),
                   jax.ShapeDtypeStruct((2, 128), _f32)],
        scratch_shapes=[pltpu.VMEM((2, 128), _f32)],
    )(x1_t, st1, g16, b16, t16, tt16, w1_t, b1_t, w2_t)


def _d3a_body(x2_ref, st_ref, g_ref, b_ref, t_ref, tt_ref, rdeg_ref,
              xo_ref, xso_ref):
    scale_t, shift_t = _bn_affine(st_ref[...], g_ref, b_ref, t_ref, tt_ref)
    xn = x2_ref[0] * scale_t + shift_t
    xo_ref[0] = xn
    xso_ref[0] = xn * rdeg_ref[0]


def _d3a_call(x2_t, st2, g16, b16, t16, tt16, rdeg_t):
    return pl.pallas_call(
        _d3a_body,
        grid=(GSTEPS,),
        in_specs=[
            _row_spec(),
            pl.BlockSpec((2, 128), lambda j: (0, 0)),
            pl.BlockSpec((1, 16), lambda j: (0, 0)),
            pl.BlockSpec((1, 16), lambda j: (0, 0)),
            pl.BlockSpec((16, 128), lambda j: (0, 0)),
            pl.BlockSpec((128, 16), lambda j: (0, 0)),
            _row_spec(),
        ],
        out_specs=[_row_spec(), _row_spec()],
        out_shape=[jax.ShapeDtypeStruct((GSTEPS, BR, 128), _f32)] * 2,
    )(x2_t, st2, g16, b16, t16, tt16, rdeg_t)


def _d3b_body(x2_ref, st_ref, g_ref, b_ref, t_ref, tt_ref, wc_ref, bc_ref,
              y_ref):
    scale_t, shift_t = _bn_affine(st_ref[...], g_ref, b_ref, t_ref, tt_ref)
    xn = x2_ref[0] * scale_t + shift_t
    y_ref[0] = (jnp.dot(xn, wc_ref[...], preferred_element_type=_f32)
                + bc_ref[0:1])


def _d3b_call(x2_t, st2, g16, b16, t16, tt16, wc_t, bc_t):
    return pl.pallas_call(
        _d3b_body,
        grid=(GSTEPS,),
        in_specs=[
            _row_spec(),
            pl.BlockSpec((2, 128), lambda j: (0, 0)),
            pl.BlockSpec((1, 16), lambda j: (0, 0)),
            pl.BlockSpec((1, 16), lambda j: (0, 0)),
            pl.BlockSpec((16, 128), lambda j: (0, 0)),
            pl.BlockSpec((128, 16), lambda j: (0, 0)),
            pl.BlockSpec((128, 320), lambda j: (0, 0)),
            pl.BlockSpec((8, 320), lambda j: (0, 0)),
        ],
        out_specs=pl.BlockSpec((1, BR, 320), lambda j: (j, 0, 0)),
        out_shape=jax.ShapeDtypeStruct((GSTEPS, BR, 320), _f32),
    )(x2_t, st2, g16, b16, t16, tt16, wc_t, bc_t)


def _kron8(w):
    return jnp.kron(jnp.eye(8, dtype=_f32), w)


def _tile_bias(b, reps, rows=8):
    return jnp.broadcast_to(jnp.tile(b, reps)[None, :], (rows, b.shape[0] * reps))


def kernel(edge_index, nodes, Wg, bg, bn1_g, bn1_b, W1, b1, W2, bn2_g, bn2_b,
           cls_W, cls_b):
    er = edge_index.reshape(2, NW, ROWS, SUB)

    r8 = jnp.repeat(jnp.eye(8, dtype=_f32), 16, axis=1)
    t16 = jnp.tile(jnp.eye(16, dtype=_f32), (1, 8))
    tt16 = t16.T

    degp = _deg_call(er)
    degp_t = degp.reshape(2, GSTEPS, BR, 8)
    x_t = nodes.reshape(GSTEPS, BR, 128)
    xs_t, rdeg_t, inv_t = _prep_call(degp_t, x_t, r8)

    y_t = None
    for i in range(2):
        xs = xs_t.reshape(N, EMB)
        ysp = _agg_call(xs, er)
        ysp_t = ysp.reshape(2, GSTEPS, BR, 128)
        x1_t, st1 = _d1_call(ysp_t, x_t, rdeg_t, inv_t,
                             _kron8(Wg[i]), _tile_bias(bg[i], 8))
        x2_t, st2 = _d2_call(x1_t, st1,
                             bn1_g[i][None, :], bn1_b[i][None, :], t16, tt16,
                             _kron8(W1[i]), _tile_bias(b1[i], 8),
                             _kron8(W2[i]))
        if i == 0:
            x_t, xs_t = _d3a_call(x2_t, st2, bn2_g[i][None, :],
                                  bn2_b[i][None, :], t16, tt16, rdeg_t)
        else:
            y_t = _d3b_call(x2_t, st2, bn2_g[i][None, :], bn2_b[i][None, :],
                            t16, tt16, _kron8(cls_W),
                            _tile_bias(cls_b, 8))
    return y_t.reshape(N, NUMCLS)


# speedup vs baseline: 86.4554x; 1.3507x over previous
import functools

import jax
import jax.numpy as jnp
from jax import lax
from jax.experimental import pallas as pl
from jax.experimental.pallas import tpu as pltpu
from jax.experimental.pallas import tpu_sc as plsc

N = 100000
E = 3200000
EMB = 16
NUMCLS = 40

NC = 2
NS = 16
NW = NC * NS

SLAB = N // NS
SLAB_A = 6256
SLAB_L = N - (NS - 1) * SLAB_A
EPT = E // NW
SUB = 80
ROWS = EPT // SUB
NSUB = 8
NGRP = ROWS // NSUB
TAIL = ROWS - NGRP * NSUB
ZR = 250

NB = N // 8
BR = 1250
GSTEPS = NB // BR

_f32 = jnp.float32


def _mesh():
    return plsc.VectorSubcoreMesh(core_axis_name="c", subcore_axis_name="s")


_SC_PARAMS = pltpu.CompilerParams(use_tc_tiling_on_sc=False)


def _deg_body(er, out_hbm, idx_v, ones_v, zbuf_v, deg_sp):
    c = lax.axis_index("c")
    s = lax.axis_index("s")
    w = c * NS + s

    def fill_ones(i, _):
        ones_v[pl.ds(i * 16, 16)] = jnp.ones((16,), _f32)
        return 0

    lax.fori_loop(0, SUB // 16, fill_ones, 0)

    def fill_zero(i, _):
        zbuf_v[pl.ds(i * 16, 16)] = jnp.zeros((16,), _f32)
        return 0

    lax.fori_loop(0, SLAB_A // 16, fill_zero, 0)

    @pl.when(s < NS - 1)
    def _():
        pltpu.sync_copy(zbuf_v, deg_sp.at[pl.ds(s * SLAB_A, SLAB_A)])

    @pl.when(s == NS - 1)
    def _():
        pltpu.sync_copy(zbuf_v.at[pl.ds(0, SLAB_L)],
                        deg_sp.at[pl.ds(s * SLAB_A, SLAB_L)])

    plsc.subcore_barrier()

    def grp(g, _):
        pltpu.sync_copy(er.at[1, w, pl.ds(g * NSUB, NSUB), :], idx_v)
        for j in range(NSUB):
            pltpu.sync_copy(ones_v, deg_sp.at[idx_v.at[j]], add=True)
        return 0

    lax.fori_loop(0, NGRP, grp, 0)
    pltpu.sync_copy(er.at[1, w, pl.ds(NGRP * NSUB, TAIL), :],
                    idx_v.at[pl.ds(0, TAIL), :])
    for j in range(TAIL):
        pltpu.sync_copy(ones_v, deg_sp.at[idx_v.at[j]], add=True)
    plsc.subcore_barrier()

    @pl.when(s < NS - 1)
    def _():
        pltpu.sync_copy(deg_sp.at[pl.ds(s * SLAB_A, SLAB_A)], zbuf_v)
        pltpu.sync_copy(zbuf_v, out_hbm.at[pl.ds(c * N + s * SLAB_A, SLAB_A)])

    @pl.when(s == NS - 1)
    def _():
        pltpu.sync_copy(deg_sp.at[pl.ds(s * SLAB_A, SLAB_L)],
                        zbuf_v.at[pl.ds(0, SLAB_L)])
        pltpu.sync_copy(zbuf_v.at[pl.ds(0, SLAB_L)],
                        out_hbm.at[pl.ds(c * N + s * SLAB_A, SLAB_L)])


def _deg_call(er):
    f = pl.kernel(
        _deg_body,
        out_type=jax.ShapeDtypeStruct((NC * N,), _f32),
        mesh=_mesh(),
        scratch_types=[
            pltpu.VMEM((NSUB, SUB), jnp.int32),
            pltpu.VMEM((SUB,), _f32),
            pltpu.VMEM((SLAB_A,), _f32),
            pltpu.VMEM_SHARED((N,), _f32),
        ],
        compiler_params=_SC_PARAMS,
    )
    return f(er)


def _agg_body(xs_hbm, er, out_hbm,
              idxs0, idxs1, idxd0, idxd1, rows0, rows1, zrow_v, ys_sp,
              gsem0, gsem1, ssem0, ssem1, isem):
    c = lax.axis_index("c")
    s = lax.axis_index("s")
    w = c * NS + s
    idxs = (idxs0, idxs1)
    idxd = (idxd0, idxd1)
    rows = (rows0, rows1)
    gsem = (gsem0, gsem1)
    ssem = (ssem0, ssem1)

    def fill_zero(i, _):
        zrow_v[i] = jnp.zeros((16,), _f32)
        return 0

    lax.fori_loop(0, ZR, fill_zero, 0)
    for k in range(SLAB // ZR):
        pltpu.sync_copy(zrow_v, ys_sp.at[pl.ds(s * SLAB + k * ZR, ZR), :])
    plsc.subcore_barrier()

    def fire_gathers(b):
        for j in range(NSUB):
            pltpu.async_copy(xs_hbm.at[idxs[b].at[j]],
                             rows[b].at[pl.ds(j * SUB, SUB), :], gsem[b])

    def wait_gathers(b):
        for j in range(NSUB):
            pltpu.make_async_copy(xs_hbm.at[idxs[b].at[j]],
                                  rows[b].at[pl.ds(j * SUB, SUB), :],
                                  gsem[b]).wait()

    def fire_scatters(b):
        for j in range(NSUB):
            pltpu.async_copy(rows[b].at[pl.ds(j * SUB, SUB), :],
                             ys_sp.at[idxd[b].at[j]], ssem[b], add=True)

    def wait_scatters(b):
        for j in range(NSUB):
            pltpu.make_async_copy(rows[b].at[pl.ds(j * SUB, SUB), :],
                                  ys_sp.at[idxd[b].at[j]], ssem[b]).wait()

    pltpu.sync_copy(er.at[0, w, pl.ds(0, NSUB), :], idxs[0])
    pltpu.sync_copy(er.at[1, w, pl.ds(0, NSUB), :], idxd[0])
    fire_gathers(0)

    def pair(gg, _):
        for b in range(2):
            g = gg * 2 + b
            nxt = g + 1

            wait_gathers(b)

            @pl.when(g > 0)
            def _():
                wait_scatters(1 - b)

            @pl.when(nxt < NGRP)
            def _():
                pltpu.async_copy(er.at[0, w, pl.ds(nxt * NSUB, NSUB), :],
                                 idxs[1 - b], isem)
                pltpu.async_copy(er.at[1, w, pl.ds(nxt * NSUB, NSUB), :],
                                 idxd[1 - b], isem)

            fire_scatters(b)

            @pl.when(nxt < NGRP)
            def _():
                pltpu.make_async_copy(er.at[0, w, pl.ds(nxt * NSUB, NSUB), :],
                                      idxs[1 - b], isem).wait()
                pltpu.make_async_copy(er.at[1, w, pl.ds(nxt * NSUB, NSUB), :],
                                      idxd[1 - b], isem).wait()
                fire_gathers(1 - b)
        return 0

    lax.fori_loop(0, NGRP // 2, pair, 0)
    wait_scatters(1)

    pltpu.sync_copy(er.at[0, w, pl.ds(NGRP * NSUB, TAIL), :],
                    idxs[0].at[pl.ds(0, TAIL), :])
    pltpu.sync_copy(er.at[1, w, pl.ds(NGRP * NSUB, TAIL), :],
                    idxd[0].at[pl.ds(0, TAIL), :])
    for j in range(TAIL):
        pltpu.async_copy(xs_hbm.at[idxs[0].at[j]],
                         rows[0].at[pl.ds(j * SUB, SUB), :], gsem[0]).wait()
    for j in range(TAIL):
        pltpu.sync_copy(rows[0].at[pl.ds(j * SUB, SUB), :],
                        ys_sp.at[idxd[0].at[j]], add=True)

    plsc.subcore_barrier()
    for k in range(SLAB // ZR):
        r0 = s * SLAB + k * ZR
        pltpu.sync_copy(ys_sp.at[pl.ds(r0, ZR), :], zrow_v)
        pltpu.sync_copy(zrow_v, out_hbm.at[c, pl.ds(r0, ZR), :])


def _agg_call(xs, er):
    f = pl.kernel(
        _agg_body,
        out_type=jax.ShapeDtypeStruct((NC, N, EMB), _f32),
        mesh=_mesh(),
        scratch_types=[
            pltpu.VMEM((NSUB, SUB), jnp.int32),
            pltpu.VMEM((NSUB, SUB), jnp.int32),
            pltpu.VMEM((NSUB, SUB), jnp.int32),
            pltpu.VMEM((NSUB, SUB), jnp.int32),
            pltpu.VMEM((NSUB * SUB, EMB), _f32),
            pltpu.VMEM((NSUB * SUB, EMB), _f32),
            pltpu.VMEM((ZR, EMB), _f32),
            pltpu.VMEM_SHARED((N, EMB), _f32),
            pltpu.SemaphoreType.DMA,
            pltpu.SemaphoreType.DMA,
            pltpu.SemaphoreType.DMA,
            pltpu.SemaphoreType.DMA,
            pltpu.SemaphoreType.DMA,
        ],
        compiler_params=_SC_PARAMS,
    )
    return f(xs, er)


def _row_spec():
    return pl.BlockSpec((1, BR, 128), lambda j: (j, 0, 0))


def _prep_body(degp_ref, x_ref, r8_ref, xs_ref, rdeg_ref, inv_ref):
    d = degp_ref[0, 0] + degp_ref[1, 0] + 1.0
    dt = jnp.dot(d, r8_ref[...], preferred_element_type=_f32)
    rdeg = lax.rsqrt(dt)
    rdeg_ref[0] = rdeg
    inv_ref[0] = 1.0 / dt
    xs_ref[0] = x_ref[0] * rdeg


def _prep_call(degp_t, x_t, r8):
    return pl.pallas_call(
        _prep_body,
        grid=(GSTEPS,),
        in_specs=[
            pl.BlockSpec((2, 1, BR, 8), lambda j: (0, j, 0, 0)),
            _row_spec(),
            pl.BlockSpec((8, 128), lambda j: (0, 0)),
        ],
        out_specs=[_row_spec(), _row_spec(), _row_spec()],
        out_shape=[jax.ShapeDtypeStruct((GSTEPS, BR, 128), _f32)] * 3,
    )(degp_t, x_t, r8)


def _d1_body(ysp_ref, x_ref, rdeg_ref, inv_ref, w_ref, b_ref,
             x1_ref, st_ref, acc):
    j = pl.program_id(0)
    x = x_ref[0]
    agg = rdeg_ref[0] * (ysp_ref[0, 0] + ysp_ref[1, 0]) + x * inv_ref[0]
    h = jnp.maximum(
        jnp.dot(agg, w_ref[...], preferred_element_type=_f32) + b_ref[0:1],
        0.0)
    x1 = h + x
    x1_ref[0] = x1

    @pl.when(j == 0)
    def _():
        acc[...] = jnp.zeros_like(acc)

    ssum = jnp.sum(x1, axis=0, keepdims=True)
    ssq = jnp.sum(x1 * x1, axis=0, keepdims=True)
    acc[...] += jnp.concatenate([ssum, ssq], axis=0)

    @pl.when(j == GSTEPS - 1)
    def _():
        st_ref[...] = acc[...]


def _d1_call(ysp_t, x_t, rdeg_t, inv_t, wg_t, bg_t):
    return pl.pallas_call(
        _d1_body,
        grid=(GSTEPS,),
        in_specs=[
            pl.BlockSpec((2, 1, BR, 128), lambda j: (0, j, 0, 0)),
            _row_spec(), _row_spec(), _row_spec(),
            pl.BlockSpec((128, 128), lambda j: (0, 0)),
            pl.BlockSpec((8, 128), lambda j: (0, 0)),
        ],
        out_specs=[_row_spec(), pl.BlockSpec((2, 128), lambda j: (0, 0))],
        out_shape=[jax.ShapeDtypeStruct((GSTEPS, BR, 128), _f32),
                   jax.ShapeDtypeStruct((2, 128), _f32)],
        scratch_shapes=[pltpu.VMEM((2, 128), _f32)],
    )(ysp_t, x_t, rdeg_t, inv_t, wg_t, bg_t)


def _bn_affine(st, g_ref, b_ref, t_ref, tt_ref):
    s16 = jnp.dot(st, tt_ref[...], preferred_element_type=_f32)
    mean = s16[0:1] / N
    var = s16[1:2] / N - mean * mean
    istd = lax.rsqrt(var + 1e-5)
    scale = istd * g_ref[...]
    shift = b_ref[...] - mean * scale
    scale_t = jnp.dot(scale, t_ref[...], preferred_element_type=_f32)
    shift_t = jnp.dot(shift, t_ref[...], preferred_element_type=_f32)
    return scale_t, shift_t


def _d2_body(x1_ref, st_ref, g_ref, b_ref, t_ref, tt_ref,
             w1_ref, b1_ref, w2_ref, x2_ref, st2_ref, acc):
    j = pl.program_id(0)
    scale_t, shift_t = _bn_affine(st_ref[...], g_ref, b_ref, t_ref, tt_ref)
    x1n = x1_ref[0] * scale_t + shift_t
    h = jnp.maximum(
        jnp.dot(x1n, w1_ref[...], preferred_element_type=_f32) + b1_ref[0:1],
        0.0)
    x2 = jnp.dot(h, w2_ref[...], preferred_element_type=_f32) + x1n
    x2_ref[0] = x2

    @pl.when(j == 0)
    def _():
        acc[...] = jnp.zeros_like(acc)

    ssum = jnp.sum(x2, axis=0, keepdims=True)
    ssq = jnp.sum(x2 * x2, axis=0, keepdims=True)
    acc[...] += jnp.concatenate([ssum, ssq], axis=0)

    @pl.when(j == GSTEPS - 1)
    def _():
        st2_ref[...] = acc[...]


def _d2_call(x1_t, st1, g16, b16, t16, tt16, w1_t, b1_t, w2_t):
    return pl.pallas_call(
        _d2_body,
        grid=(GSTEPS,),
        in_specs=[
            _row_spec(),
            pl.BlockSpec((2, 128), lambda j: (0, 0)),
            pl.BlockSpec((1, 16), lambda j: (0, 0)),
            pl.BlockSpec((1, 16), lambda j: (0, 0)),
            pl.BlockSpec((16, 128), lambda j: (0, 0)),
            pl.BlockSpec((128, 16), lambda j: (0, 0)),
            pl.BlockSpec((128, 512), lambda j: (0, 0)),
            pl.BlockSpec((8, 512), lambda j: (0, 0)),
            pl.BlockSpec((512, 128), lambda j: (0, 0)),
        ],
        out_specs=[_row_spec(), pl.BlockSpec((2, 128), lambda j: (0, 0))],
        out_shape=[jax.ShapeDtypeStruct((GSTEPS, BR, 128), _f32),
                   jax.ShapeDtypeStruct((2, 128), _f32)],
        scratch_shapes=[pltpu.VMEM((2, 128), _f32)],
    )(x1_t, st1, g16, b16, t16, tt16, w1_t, b1_t, w2_t)


def _d3a_body(x2_ref, st_ref, g_ref, b_ref, t_ref, tt_ref, rdeg_ref,
              xo_ref, xso_ref):
    scale_t, shift_t = _bn_affine(st_ref[...], g_ref, b_ref, t_ref, tt_ref)
    xn = x2_ref[0] * scale_t + shift_t
    xo_ref[0] = xn
    xso_ref[0] = xn * rdeg_ref[0]


def _d3a_call(x2_t, st2, g16, b16, t16, tt16, rdeg_t):
    return pl.pallas_call(
        _d3a_body,
        grid=(GSTEPS,),
        in_specs=[
            _row_spec(),
            pl.BlockSpec((2, 128), lambda j: (0, 0)),
            pl.BlockSpec((1, 16), lambda j: (0, 0)),
            pl.BlockSpec((1, 16), lambda j: (0, 0)),
            pl.BlockSpec((16, 128), lambda j: (0, 0)),
            pl.BlockSpec((128, 16), lambda j: (0, 0)),
            _row_spec(),
        ],
        out_specs=[_row_spec(), _row_spec()],
        out_shape=[jax.ShapeDtypeStruct((GSTEPS, BR, 128), _f32)] * 2,
    )(x2_t, st2, g16, b16, t16, tt16, rdeg_t)


def _d3b_body(x2_ref, st_ref, g_ref, b_ref, t_ref, tt_ref, wc_ref, bc_ref,
              y_ref):
    scale_t, shift_t = _bn_affine(st_ref[...], g_ref, b_ref, t_ref, tt_ref)
    xn = x2_ref[0] * scale_t + shift_t
    y_ref[0] = (jnp.dot(xn, wc_ref[...], preferred_element_type=_f32)
                + bc_ref[0:1])


def _d3b_call(x2_t, st2, g16, b16, t16, tt16, wc_t, bc_t):
    return pl.pallas_call(
        _d3b_body,
        grid=(GSTEPS,),
        in_specs=[
            _row_spec(),
            pl.BlockSpec((2, 128), lambda j: (0, 0)),
            pl.BlockSpec((1, 16), lambda j: (0, 0)),
            pl.BlockSpec((1, 16), lambda j: (0, 0)),
            pl.BlockSpec((16, 128), lambda j: (0, 0)),
            pl.BlockSpec((128, 16), lambda j: (0, 0)),
            pl.BlockSpec((128, 320), lambda j: (0, 0)),
            pl.BlockSpec((8, 320), lambda j: (0, 0)),
        ],
        out_specs=pl.BlockSpec((1, BR, 320), lambda j: (j, 0, 0)),
        out_shape=jax.ShapeDtypeStruct((GSTEPS, BR, 320), _f32),
    )(x2_t, st2, g16, b16, t16, tt16, wc_t, bc_t)


def _kron8(w):
    return jnp.kron(jnp.eye(8, dtype=_f32), w)


def _tile_bias(b, reps, rows=8):
    return jnp.broadcast_to(jnp.tile(b, reps)[None, :], (rows, b.shape[0] * reps))


def kernel(edge_index, nodes, Wg, bg, bn1_g, bn1_b, W1, b1, W2, bn2_g, bn2_b,
           cls_W, cls_b):
    er = edge_index.reshape(2, NW, ROWS, SUB)

    r8 = jnp.repeat(jnp.eye(8, dtype=_f32), 16, axis=1)
    t16 = jnp.tile(jnp.eye(16, dtype=_f32), (1, 8))
    tt16 = t16.T

    degp = _deg_call(er)
    degp_t = degp.reshape(2, GSTEPS, BR, 8)
    x_t = nodes.reshape(GSTEPS, BR, 128)
    xs_t, rdeg_t, inv_t = _prep_call(degp_t, x_t, r8)

    y_t = None
    for i in range(2):
        xs = xs_t.reshape(N, EMB)
        ysp = _agg_call(xs, er)
        ysp_t = ysp.reshape(2, GSTEPS, BR, 128)
        x1_t, st1 = _d1_call(ysp_t, x_t, rdeg_t, inv_t,
                             _kron8(Wg[i]), _tile_bias(bg[i], 8))
        x2_t, st2 = _d2_call(x1_t, st1,
                             bn1_g[i][None, :], bn1_b[i][None, :], t16, tt16,
                             _kron8(W1[i]), _tile_bias(b1[i], 8),
                             _kron8(W2[i]))
        if i == 0:
            x_t, xs_t = _d3a_call(x2_t, st2, bn2_g[i][None, :],
                                  bn2_b[i][None, :], t16, tt16, rdeg_t)
        else:
            y_t = _d3b_call(x2_t, st2, bn2_g[i][None, :], bn2_b[i][None, :],
                            t16, tt16, _kron8(cls_W),
                            _tile_bias(cls_b, 8))
    return y_t.reshape(N, NUMCLS)


# speedup vs baseline: 102.5221x; 1.1858x over previous
import functools

import jax
import jax.numpy as jnp
from jax import lax
from jax.experimental import pallas as pl
from jax.experimental.pallas import tpu as pltpu
from jax.experimental.pallas import tpu_sc as plsc

N = 100000
E = 3200000
EMB = 16
NUMCLS = 40

NC = 2
NS = 16
NW = NC * NS

SLAB = N // NS
SLAB_A = 6256
SLAB_L = N - (NS - 1) * SLAB_A
EPT = E // NW
SUB = 80
ROWS = EPT // SUB
NSUB = 10
NGRP = ROWS // NSUB
ZR = 125

NB = N // 8
BR = 1250
GSTEPS = NB // BR

_f32 = jnp.float32


def _mesh():
    return plsc.VectorSubcoreMesh(core_axis_name="c", subcore_axis_name="s")


_SC_PARAMS = pltpu.CompilerParams(use_tc_tiling_on_sc=False)


def _deg_body(er, out_hbm, idx0, idx1, ones_v, zbuf_v, deg_sp,
              ssem0, ssem1, isem):
    c = lax.axis_index("c")
    s = lax.axis_index("s")
    w = c * NS + s
    idx = (idx0, idx1)
    ssem = (ssem0, ssem1)

    def fill_ones(i, _):
        ones_v[pl.ds(i * 16, 16)] = jnp.ones((16,), _f32)
        return 0

    lax.fori_loop(0, SUB // 16, fill_ones, 0)

    def fill_zero(i, _):
        zbuf_v[pl.ds(i * 16, 16)] = jnp.zeros((16,), _f32)
        return 0

    lax.fori_loop(0, SLAB_A // 16, fill_zero, 0)

    @pl.when(s < NS - 1)
    def _():
        pltpu.sync_copy(zbuf_v, deg_sp.at[pl.ds(s * SLAB_A, SLAB_A)])

    @pl.when(s == NS - 1)
    def _():
        pltpu.sync_copy(zbuf_v.at[pl.ds(0, SLAB_L)],
                        deg_sp.at[pl.ds(s * SLAB_A, SLAB_L)])

    plsc.subcore_barrier()

    def fire_scatters(b):
        for j in range(NSUB):
            pltpu.async_copy(ones_v, deg_sp.at[idx[b].at[j]], ssem[b],
                             add=True)

    def wait_scatters(b):
        for j in range(NSUB):
            pltpu.make_async_copy(ones_v, deg_sp.at[idx[b].at[j]],
                                  ssem[b]).wait()

    pltpu.sync_copy(er.at[1, w, pl.ds(0, NSUB), :], idx[0])

    def step(g, b):
        nxt = g + 1

        @pl.when(g > 0)
        def _():
            wait_scatters(1 - b)

        @pl.when(nxt < NGRP)
        def _():
            pltpu.async_copy(er.at[1, w, pl.ds(nxt * NSUB, NSUB), :],
                             idx[1 - b], isem)

        fire_scatters(b)

        @pl.when(nxt < NGRP)
        def _():
            pltpu.make_async_copy(er.at[1, w, pl.ds(nxt * NSUB, NSUB), :],
                                  idx[1 - b], isem).wait()

    def pair(gg, _):
        step(gg * 2, 0)
        step(gg * 2 + 1, 1)
        return 0

    lax.fori_loop(0, NGRP // 2, pair, 0)
    step(NGRP - 1, (NGRP - 1) % 2)
    wait_scatters((NGRP - 1) % 2)
    plsc.subcore_barrier()

    @pl.when(s < NS - 1)
    def _():
        pltpu.sync_copy(deg_sp.at[pl.ds(s * SLAB_A, SLAB_A)], zbuf_v)
        pltpu.sync_copy(zbuf_v, out_hbm.at[pl.ds(c * N + s * SLAB_A, SLAB_A)])

    @pl.when(s == NS - 1)
    def _():
        pltpu.sync_copy(deg_sp.at[pl.ds(s * SLAB_A, SLAB_L)],
                        zbuf_v.at[pl.ds(0, SLAB_L)])
        pltpu.sync_copy(zbuf_v.at[pl.ds(0, SLAB_L)],
                        out_hbm.at[pl.ds(c * N + s * SLAB_A, SLAB_L)])


def _deg_call(er):
    f = pl.kernel(
        _deg_body,
        out_type=jax.ShapeDtypeStruct((NC * N,), _f32),
        mesh=_mesh(),
        scratch_types=[
            pltpu.VMEM((NSUB, SUB), jnp.int32),
            pltpu.VMEM((NSUB, SUB), jnp.int32),
            pltpu.VMEM((SUB,), _f32),
            pltpu.VMEM((SLAB_A,), _f32),
            pltpu.VMEM_SHARED((N,), _f32),
            pltpu.SemaphoreType.DMA,
            pltpu.SemaphoreType.DMA,
            pltpu.SemaphoreType.DMA,
        ],
        compiler_params=_SC_PARAMS,
    )
    return f(er)


def _agg_body(xs_hbm, er, out_hbm,
              idxs0, idxs1, idxd0, idxd1, rows0, rows1, zrow_v, ys_sp,
              gsem0, gsem1, ssem0, ssem1, isem):
    c = lax.axis_index("c")
    s = lax.axis_index("s")
    w = c * NS + s
    idxs = (idxs0, idxs1)
    idxd = (idxd0, idxd1)
    rows = (rows0, rows1)
    gsem = (gsem0, gsem1)
    ssem = (ssem0, ssem1)

    def fill_zero(i, _):
        zrow_v[i] = jnp.zeros((16,), _f32)
        return 0

    lax.fori_loop(0, ZR, fill_zero, 0)
    for k in range(SLAB // ZR):
        pltpu.sync_copy(zrow_v, ys_sp.at[pl.ds(s * SLAB + k * ZR, ZR), :])
    plsc.subcore_barrier()

    def fire_gathers(b):
        for j in range(NSUB):
            pltpu.async_copy(xs_hbm.at[idxs[b].at[j]],
                             rows[b].at[pl.ds(j * SUB, SUB), :], gsem[b])

    def wait_gathers(b):
        for j in range(NSUB):
            pltpu.make_async_copy(xs_hbm.at[idxs[b].at[j]],
                                  rows[b].at[pl.ds(j * SUB, SUB), :],
                                  gsem[b]).wait()

    def fire_scatters(b):
        for j in range(NSUB):
            pltpu.async_copy(rows[b].at[pl.ds(j * SUB, SUB), :],
                             ys_sp.at[idxd[b].at[j]], ssem[b], add=True)

    def wait_scatters(b):
        for j in range(NSUB):
            pltpu.make_async_copy(rows[b].at[pl.ds(j * SUB, SUB), :],
                                  ys_sp.at[idxd[b].at[j]], ssem[b]).wait()

    pltpu.sync_copy(er.at[0, w, pl.ds(0, NSUB), :], idxs[0])
    pltpu.sync_copy(er.at[1, w, pl.ds(0, NSUB), :], idxd[0])
    fire_gathers(0)

    def step(g, b):
        nxt = g + 1

        wait_gathers(b)

        @pl.when(g > 0)
        def _():
            wait_scatters(1 - b)

        @pl.when(nxt < NGRP)
        def _():
            pltpu.async_copy(er.at[0, w, pl.ds(nxt * NSUB, NSUB), :],
                             idxs[1 - b], isem)
            pltpu.async_copy(er.at[1, w, pl.ds(nxt * NSUB, NSUB), :],
                             idxd[1 - b], isem)

        fire_scatters(b)

        @pl.when(nxt < NGRP)
        def _():
            pltpu.make_async_copy(er.at[0, w, pl.ds(nxt * NSUB, NSUB), :],
                                  idxs[1 - b], isem).wait()
            pltpu.make_async_copy(er.at[1, w, pl.ds(nxt * NSUB, NSUB), :],
                                  idxd[1 - b], isem).wait()
            fire_gathers(1 - b)

    def pair(gg, _):
        step(gg * 2, 0)
        step(gg * 2 + 1, 1)
        return 0

    lax.fori_loop(0, NGRP // 2, pair, 0)
    step(NGRP - 1, (NGRP - 1) % 2)
    wait_scatters((NGRP - 1) % 2)

    plsc.subcore_barrier()
    for k in range(SLAB // ZR):
        r0 = s * SLAB + k * ZR
        pltpu.sync_copy(ys_sp.at[pl.ds(r0, ZR), :], zrow_v)
        pltpu.sync_copy(zrow_v, out_hbm.at[c, pl.ds(r0, ZR), :])


def _agg_call(xs, er):
    f = pl.kernel(
        _agg_body,
        out_type=jax.ShapeDtypeStruct((NC, N, EMB), _f32),
        mesh=_mesh(),
        scratch_types=[
            pltpu.VMEM((NSUB, SUB), jnp.int32),
            pltpu.VMEM((NSUB, SUB), jnp.int32),
            pltpu.VMEM((NSUB, SUB), jnp.int32),
            pltpu.VMEM((NSUB, SUB), jnp.int32),
            pltpu.VMEM((NSUB * SUB, EMB), _f32),
            pltpu.VMEM((NSUB * SUB, EMB), _f32),
            pltpu.VMEM((ZR, EMB), _f32),
            pltpu.VMEM_SHARED((N, EMB), _f32),
            pltpu.SemaphoreType.DMA,
            pltpu.SemaphoreType.DMA,
            pltpu.SemaphoreType.DMA,
            pltpu.SemaphoreType.DMA,
            pltpu.SemaphoreType.DMA,
        ],
        compiler_params=_SC_PARAMS,
    )
    return f(xs, er)


def _row_spec():
    return pl.BlockSpec((1, BR, 128), lambda j: (j, 0, 0))


def _prep_body(degp_ref, x_ref, r8_ref, xs_ref, rdeg_ref, inv_ref):
    d = degp_ref[0, 0] + degp_ref[1, 0] + 1.0
    dt = jnp.dot(d, r8_ref[...], preferred_element_type=_f32)
    rdeg = lax.rsqrt(dt)
    rdeg_ref[0] = rdeg
    inv_ref[0] = 1.0 / dt
    xs_ref[0] = x_ref[0] * rdeg


def _prep_call(degp_t, x_t, r8):
    return pl.pallas_call(
        _prep_body,
        grid=(GSTEPS,),
        in_specs=[
            pl.BlockSpec((2, 1, BR, 8), lambda j: (0, j, 0, 0)),
            _row_spec(),
            pl.BlockSpec((8, 128), lambda j: (0, 0)),
        ],
        out_specs=[_row_spec(), _row_spec(), _row_spec()],
        out_shape=[jax.ShapeDtypeStruct((GSTEPS, BR, 128), _f32)] * 3,
    )(degp_t, x_t, r8)


def _d1_body(ysp_ref, x_ref, rdeg_ref, inv_ref, w_ref, b_ref,
             x1_ref, st_ref, acc):
    j = pl.program_id(0)
    x = x_ref[0]
    agg = rdeg_ref[0] * (ysp_ref[0, 0] + ysp_ref[1, 0]) + x * inv_ref[0]
    h = jnp.maximum(
        jnp.dot(agg, w_ref[...], preferred_element_type=_f32) + b_ref[0:1],
        0.0)
    x1 = h + x
    x1_ref[0] = x1

    @pl.when(j == 0)
    def _():
        acc[...] = jnp.zeros_like(acc)

    ssum = jnp.sum(x1, axis=0, keepdims=True)
    ssq = jnp.sum(x1 * x1, axis=0, keepdims=True)
    acc[...] += jnp.concatenate([ssum, ssq], axis=0)

    @pl.when(j == GSTEPS - 1)
    def _():
        st_ref[...] = acc[...]


def _d1_call(ysp_t, x_t, rdeg_t, inv_t, wg_t, bg_t):
    return pl.pallas_call(
        _d1_body,
        grid=(GSTEPS,),
        in_specs=[
            pl.BlockSpec((2, 1, BR, 128), lambda j: (0, j, 0, 0)),
            _row_spec(), _row_spec(), _row_spec(),
            pl.BlockSpec((128, 128), lambda j: (0, 0)),
            pl.BlockSpec((8, 128), lambda j: (0, 0)),
        ],
        out_specs=[_row_spec(), pl.BlockSpec((2, 128), lambda j: (0, 0))],
        out_shape=[jax.ShapeDtypeStruct((GSTEPS, BR, 128), _f32),
                   jax.ShapeDtypeStruct((2, 128), _f32)],
        scratch_shapes=[pltpu.VMEM((2, 128), _f32)],
    )(ysp_t, x_t, rdeg_t, inv_t, wg_t, bg_t)


def _bn_affine(st, g_ref, b_ref, t_ref, tt_ref):
    s16 = jnp.dot(st, tt_ref[...], preferred_element_type=_f32)
    mean = s16[0:1] / N
    var = s16[1:2] / N - mean * mean
    istd = lax.rsqrt(var + 1e-5)
    scale = istd * g_ref[...]
    shift = b_ref[...] - mean * scale
    scale_t = jnp.dot(scale, t_ref[...], preferred_element_type=_f32)
    shift_t = jnp.dot(shift, t_ref[...], preferred_element_type=_f32)
    return scale_t, shift_t


def _d2_body(x1_ref, st_ref, g_ref, b_ref, t_ref, tt_ref,
             w1_ref, b1_ref, w2_ref, x2_ref, st2_ref, acc):
    j = pl.program_id(0)
    scale_t, shift_t = _bn_affine(st_ref[...], g_ref, b_ref, t_ref, tt_ref)
    x1n = x1_ref[0] * scale_t + shift_t
    h = jnp.maximum(
        jnp.dot(x1n, w1_ref[...], preferred_element_type=_f32) + b1_ref[0:1],
        0.0)
    x2 = jnp.dot(h, w2_ref[...], preferred_element_type=_f32) + x1n
    x2_ref[0] = x2

    @pl.when(j == 0)
    def _():
        acc[...] = jnp.zeros_like(acc)

    ssum = jnp.sum(x2, axis=0, keepdims=True)
    ssq = jnp.sum(x2 * x2, axis=0, keepdims=True)
    acc[...] += jnp.concatenate([ssum, ssq], axis=0)

    @pl.when(j == GSTEPS - 1)
    def _():
        st2_ref[...] = acc[...]


def _d2_call(x1_t, st1, g16, b16, t16, tt16, w1_t, b1_t, w2_t):
    return pl.pallas_call(
        _d2_body,
        grid=(GSTEPS,),
        in_specs=[
            _row_spec(),
            pl.BlockSpec((2, 128), lambda j: (0, 0)),
            pl.BlockSpec((1, 16), lambda j: (0, 0)),
            pl.BlockSpec((1, 16), lambda j: (0, 0)),
            pl.BlockSpec((16, 128), lambda j: (0, 0)),
            pl.BlockSpec((128, 16), lambda j: (0, 0)),
            pl.BlockSpec((128, 512), lambda j: (0, 0)),
            pl.BlockSpec((8, 512), lambda j: (0, 0)),
            pl.BlockSpec((512, 128), lambda j: (0, 0)),
        ],
        out_specs=[_row_spec(), pl.BlockSpec((2, 128), lambda j: (0, 0))],
        out_shape=[jax.ShapeDtypeStruct((GSTEPS, BR, 128), _f32),
                   jax.ShapeDtypeStruct((2, 128), _f32)],
        scratch_shapes=[pltpu.VMEM((2, 128), _f32)],
    )(x1_t, st1, g16, b16, t16, tt16, w1_t, b1_t, w2_t)


def _d3a_body(x2_ref, st_ref, g_ref, b_ref, t_ref, tt_ref, rdeg_ref,
              xo_ref, xso_ref):
    scale_t, shift_t = _bn_affine(st_ref[...], g_ref, b_ref, t_ref, tt_ref)
    xn = x2_ref[0] * scale_t + shift_t
    xo_ref[0] = xn
    xso_ref[0] = xn * rdeg_ref[0]


def _d3a_call(x2_t, st2, g16, b16, t16, tt16, rdeg_t):
    return pl.pallas_call(
        _d3a_body,
        grid=(GSTEPS,),
        in_specs=[
            _row_spec(),
            pl.BlockSpec((2, 128), lambda j: (0, 0)),
            pl.BlockSpec((1, 16), lambda j: (0, 0)),
            pl.BlockSpec((1, 16), lambda j: (0, 0)),
            pl.BlockSpec((16, 128), lambda j: (0, 0)),
            pl.BlockSpec((128, 16), lambda j: (0, 0)),
            _row_spec(),
        ],
        out_specs=[_row_spec(), _row_spec()],
        out_shape=[jax.ShapeDtypeStruct((GSTEPS, BR, 128), _f32)] * 2,
    )(x2_t, st2, g16, b16, t16, tt16, rdeg_t)


def _d3b_body(x2_ref, st_ref, g_ref, b_ref, t_ref, tt_ref, wc_ref, bc_ref,
              y_ref):
    scale_t, shift_t = _bn_affine(st_ref[...], g_ref, b_ref, t_ref, tt_ref)
    xn = x2_ref[0] * scale_t + shift_t
    y_ref[0] = (jnp.dot(xn, wc_ref[...], preferred_element_type=_f32)
                + bc_ref[0:1])


def _d3b_call(x2_t, st2, g16, b16, t16, tt16, wc_t, bc_t):
    return pl.pallas_call(
        _d3b_body,
        grid=(GSTEPS,),
        in_specs=[
            _row_spec(),
            pl.BlockSpec((2, 128), lambda j: (0, 0)),
            pl.BlockSpec((1, 16), lambda j: (0, 0)),
            pl.BlockSpec((1, 16), lambda j: (0, 0)),
            pl.BlockSpec((16, 128), lambda j: (0, 0)),
            pl.BlockSpec((128, 16), lambda j: (0, 0)),
            pl.BlockSpec((128, 320), lambda j: (0, 0)),
            pl.BlockSpec((8, 320), lambda j: (0, 0)),
        ],
        out_specs=pl.BlockSpec((1, BR, 320), lambda j: (j, 0, 0)),
        out_shape=jax.ShapeDtypeStruct((GSTEPS, BR, 320), _f32),
    )(x2_t, st2, g16, b16, t16, tt16, wc_t, bc_t)


def _kron8(w):
    return jnp.kron(jnp.eye(8, dtype=_f32), w)


def _tile_bias(b, reps, rows=8):
    return jnp.broadcast_to(jnp.tile(b, reps)[None, :], (rows, b.shape[0] * reps))


def kernel(edge_index, nodes, Wg, bg, bn1_g, bn1_b, W1, b1, W2, bn2_g, bn2_b,
           cls_W, cls_b):
    er = edge_index.reshape(2, NW, ROWS, SUB)

    r8 = jnp.repeat(jnp.eye(8, dtype=_f32), 16, axis=1)
    t16 = jnp.tile(jnp.eye(16, dtype=_f32), (1, 8))
    tt16 = t16.T

    degp = _deg_call(er)
    degp_t = degp.reshape(2, GSTEPS, BR, 8)
    x_t = nodes.reshape(GSTEPS, BR, 128)
    xs_t, rdeg_t, inv_t = _prep_call(degp_t, x_t, r8)

    y_t = None
    for i in range(2):
        xs = xs_t.reshape(N, EMB)
        ysp = _agg_call(xs, er)
        ysp_t = ysp.reshape(2, GSTEPS, BR, 128)
        x1_t, st1 = _d1_call(ysp_t, x_t, rdeg_t, inv_t,
                             _kron8(Wg[i]), _tile_bias(bg[i], 8))
        x2_t, st2 = _d2_call(x1_t, st1,
                             bn1_g[i][None, :], bn1_b[i][None, :], t16, tt16,
                             _kron8(W1[i]), _tile_bias(b1[i], 8),
                             _kron8(W2[i]))
        if i == 0:
            x_t, xs_t = _d3a_call(x2_t, st2, bn2_g[i][None, :],
                                  bn2_b[i][None, :], t16, tt16, rdeg_t)
        else:
            y_t = _d3b_call(x2_t, st2, bn2_g[i][None, :], bn2_b[i][None, :],
                            t16, tt16, _kron8(cls_W),
                            _tile_bias(cls_b, 8))
    return y_t.reshape(N, NUMCLS)


# speedup vs baseline: 103.3458x; 1.0080x over previous
import functools

import jax
import jax.numpy as jnp
from jax import lax
from jax.experimental import pallas as pl
from jax.experimental.pallas import tpu as pltpu
from jax.experimental.pallas import tpu_sc as plsc

N = 100000
E = 3200000
EMB = 16
NUMCLS = 40

NC = 2
NS = 16
NW = NC * NS

SLAB = N // NS
SLAB_A = 6256
SLAB_L = N - (NS - 1) * SLAB_A
EPT = E // NW
SUB = 80
ROWS = EPT // SUB
NSUB = 10
NGRP = ROWS // NSUB
ZR = 125

NB = N // 8
BR = 1250
GSTEPS = NB // BR

_f32 = jnp.float32


def _mesh():
    return plsc.VectorSubcoreMesh(core_axis_name="c", subcore_axis_name="s")


_SC_PARAMS = pltpu.CompilerParams(use_tc_tiling_on_sc=False)


def _deg_body(er, out_hbm, idx0, idx1, ones_v, zbuf_v, deg_sp,
              ssem0, ssem1, isem):
    c = lax.axis_index("c")
    s = lax.axis_index("s")
    w = c * NS + s
    idx = (idx0, idx1)
    ssem = (ssem0, ssem1)

    def fill_ones(i, _):
        ones_v[pl.ds(i * 16, 16)] = jnp.ones((16,), _f32)
        return 0

    lax.fori_loop(0, SUB // 16, fill_ones, 0)

    def fill_zero(i, _):
        zbuf_v[pl.ds(i * 16, 16)] = jnp.zeros((16,), _f32)
        return 0

    lax.fori_loop(0, SLAB_A // 16, fill_zero, 0)

    @pl.when(s < NS - 1)
    def _():
        pltpu.sync_copy(zbuf_v, deg_sp.at[pl.ds(s * SLAB_A, SLAB_A)])

    @pl.when(s == NS - 1)
    def _():
        pltpu.sync_copy(zbuf_v.at[pl.ds(0, SLAB_L)],
                        deg_sp.at[pl.ds(s * SLAB_A, SLAB_L)])

    plsc.subcore_barrier()

    def fire_scatters(b):
        for j in range(NSUB):
            pltpu.async_copy(ones_v, deg_sp.at[idx[b].at[j]], ssem[b],
                             add=True)

    def wait_scatters(b):
        pltpu.make_async_copy(er.at[1, w, pl.ds(0, NSUB), :], idx[b],
                              ssem[b]).wait()

    pltpu.sync_copy(er.at[1, w, pl.ds(0, NSUB), :], idx[0])

    def step(g, b):
        nxt = g + 1

        @pl.when(g > 0)
        def _():
            wait_scatters(1 - b)

        @pl.when(nxt < NGRP)
        def _():
            pltpu.async_copy(er.at[1, w, pl.ds(nxt * NSUB, NSUB), :],
                             idx[1 - b], isem)

        fire_scatters(b)

        @pl.when(nxt < NGRP)
        def _():
            pltpu.make_async_copy(er.at[1, w, pl.ds(nxt * NSUB, NSUB), :],
                                  idx[1 - b], isem).wait()

    def pair(gg, _):
        step(gg * 2, 0)
        step(gg * 2 + 1, 1)
        return 0

    lax.fori_loop(0, NGRP // 2, pair, 0)
    step(NGRP - 1, (NGRP - 1) % 2)
    wait_scatters((NGRP - 1) % 2)
    plsc.subcore_barrier()

    @pl.when(s < NS - 1)
    def _():
        pltpu.sync_copy(deg_sp.at[pl.ds(s * SLAB_A, SLAB_A)], zbuf_v)
        pltpu.sync_copy(zbuf_v, out_hbm.at[pl.ds(c * N + s * SLAB_A, SLAB_A)])

    @pl.when(s == NS - 1)
    def _():
        pltpu.sync_copy(deg_sp.at[pl.ds(s * SLAB_A, SLAB_L)],
                        zbuf_v.at[pl.ds(0, SLAB_L)])
        pltpu.sync_copy(zbuf_v.at[pl.ds(0, SLAB_L)],
                        out_hbm.at[pl.ds(c * N + s * SLAB_A, SLAB_L)])


def _deg_call(er):
    f = pl.kernel(
        _deg_body,
        out_type=jax.ShapeDtypeStruct((NC * N,), _f32),
        mesh=_mesh(),
        scratch_types=[
            pltpu.VMEM((NSUB, SUB), jnp.int32),
            pltpu.VMEM((NSUB, SUB), jnp.int32),
            pltpu.VMEM((SUB,), _f32),
            pltpu.VMEM((SLAB_A,), _f32),
            pltpu.VMEM_SHARED((N,), _f32),
            pltpu.SemaphoreType.DMA,
            pltpu.SemaphoreType.DMA,
            pltpu.SemaphoreType.DMA,
        ],
        compiler_params=_SC_PARAMS,
    )
    return f(er)


def _agg_body(xs_hbm, er, out_hbm,
              idxs0, idxs1, idxd0, idxd1, rows0, rows1, zrow_v, ys_sp,
              gsem0, gsem1, ssem0, ssem1, isem):
    c = lax.axis_index("c")
    s = lax.axis_index("s")
    w = c * NS + s
    idxs = (idxs0, idxs1)
    idxd = (idxd0, idxd1)
    rows = (rows0, rows1)
    gsem = (gsem0, gsem1)
    ssem = (ssem0, ssem1)

    def fill_zero(i, _):
        zrow_v[i] = jnp.zeros((16,), _f32)
        return 0

    lax.fori_loop(0, ZR, fill_zero, 0)
    for k in range(SLAB // ZR):
        pltpu.sync_copy(zrow_v, ys_sp.at[pl.ds(s * SLAB + k * ZR, ZR), :])
    plsc.subcore_barrier()

    def fire_gathers(b):
        for j in range(NSUB):
            pltpu.async_copy(xs_hbm.at[idxs[b].at[j]],
                             rows[b].at[pl.ds(j * SUB, SUB), :], gsem[b])

    def wait_gathers(b):
        pltpu.make_async_copy(xs_hbm.at[pl.ds(0, NSUB * SUB), :],
                              rows[b], gsem[b]).wait()

    def fire_scatters(b):
        for j in range(NSUB):
            pltpu.async_copy(rows[b].at[pl.ds(j * SUB, SUB), :],
                             ys_sp.at[idxd[b].at[j]], ssem[b], add=True)

    def wait_scatters(b):
        pltpu.make_async_copy(xs_hbm.at[pl.ds(0, NSUB * SUB), :],
                              rows[b], ssem[b]).wait()

    pltpu.sync_copy(er.at[0, w, pl.ds(0, NSUB), :], idxs[0])
    pltpu.sync_copy(er.at[1, w, pl.ds(0, NSUB), :], idxd[0])
    fire_gathers(0)

    def step(g, b):
        nxt = g + 1

        wait_gathers(b)

        @pl.when(g > 0)
        def _():
            wait_scatters(1 - b)

        @pl.when(nxt < NGRP)
        def _():
            pltpu.async_copy(er.at[0, w, pl.ds(nxt * NSUB, NSUB), :],
                             idxs[1 - b], isem)
            pltpu.async_copy(er.at[1, w, pl.ds(nxt * NSUB, NSUB), :],
                             idxd[1 - b], isem)

        fire_scatters(b)

        @pl.when(nxt < NGRP)
        def _():
            pltpu.make_async_copy(er.at[0, w, pl.ds(nxt * NSUB, NSUB), :],
                                  idxs[1 - b], isem).wait()
            pltpu.make_async_copy(er.at[1, w, pl.ds(nxt * NSUB, NSUB), :],
                                  idxd[1 - b], isem).wait()
            fire_gathers(1 - b)

    def pair(gg, _):
        step(gg * 2, 0)
        step(gg * 2 + 1, 1)
        return 0

    lax.fori_loop(0, NGRP // 2, pair, 0)
    step(NGRP - 1, (NGRP - 1) % 2)
    wait_scatters((NGRP - 1) % 2)

    plsc.subcore_barrier()
    for k in range(SLAB // ZR):
        r0 = s * SLAB + k * ZR
        pltpu.sync_copy(ys_sp.at[pl.ds(r0, ZR), :], zrow_v)
        pltpu.sync_copy(zrow_v, out_hbm.at[c, pl.ds(r0, ZR), :])


def _agg_call(xs, er):
    f = pl.kernel(
        _agg_body,
        out_type=jax.ShapeDtypeStruct((NC, N, EMB), _f32),
        mesh=_mesh(),
        scratch_types=[
            pltpu.VMEM((NSUB, SUB), jnp.int32),
            pltpu.VMEM((NSUB, SUB), jnp.int32),
            pltpu.VMEM((NSUB, SUB), jnp.int32),
            pltpu.VMEM((NSUB, SUB), jnp.int32),
            pltpu.VMEM((NSUB * SUB, EMB), _f32),
            pltpu.VMEM((NSUB * SUB, EMB), _f32),
            pltpu.VMEM((ZR, EMB), _f32),
            pltpu.VMEM_SHARED((N, EMB), _f32),
            pltpu.SemaphoreType.DMA,
            pltpu.SemaphoreType.DMA,
            pltpu.SemaphoreType.DMA,
            pltpu.SemaphoreType.DMA,
            pltpu.SemaphoreType.DMA,
        ],
        compiler_params=_SC_PARAMS,
    )
    return f(xs, er)


def _row_spec():
    return pl.BlockSpec((1, BR, 128), lambda j: (j, 0, 0))


def _prep_body(degp_ref, x_ref, r8_ref, xs_ref, rdeg_ref, inv_ref):
    d = degp_ref[0, 0] + degp_ref[1, 0] + 1.0
    dt = jnp.dot(d, r8_ref[...], preferred_element_type=_f32)
    rdeg = lax.rsqrt(dt)
    rdeg_ref[0] = rdeg
    inv_ref[0] = 1.0 / dt
    xs_ref[0] = x_ref[0] * rdeg


def _prep_call(degp_t, x_t, r8):
    return pl.pallas_call(
        _prep_body,
        grid=(GSTEPS,),
        in_specs=[
            pl.BlockSpec((2, 1, BR, 8), lambda j: (0, j, 0, 0)),
            _row_spec(),
            pl.BlockSpec((8, 128), lambda j: (0, 0)),
        ],
        out_specs=[_row_spec(), _row_spec(), _row_spec()],
        out_shape=[jax.ShapeDtypeStruct((GSTEPS, BR, 128), _f32)] * 3,
    )(degp_t, x_t, r8)


def _d1_body(ysp_ref, x_ref, rdeg_ref, inv_ref, w_ref, b_ref,
             x1_ref, st_ref, acc):
    j = pl.program_id(0)
    x = x_ref[0]
    agg = rdeg_ref[0] * (ysp_ref[0, 0] + ysp_ref[1, 0]) + x * inv_ref[0]
    h = jnp.maximum(
        jnp.dot(agg, w_ref[...], preferred_element_type=_f32) + b_ref[0:1],
        0.0)
    x1 = h + x
    x1_ref[0] = x1

    @pl.when(j == 0)
    def _():
        acc[...] = jnp.zeros_like(acc)

    ssum = jnp.sum(x1, axis=0, keepdims=True)
    ssq = jnp.sum(x1 * x1, axis=0, keepdims=True)
    acc[...] += jnp.concatenate([ssum, ssq], axis=0)

    @pl.when(j == GSTEPS - 1)
    def _():
        st_ref[...] = acc[...]


def _d1_call(ysp_t, x_t, rdeg_t, inv_t, wg_t, bg_t):
    return pl.pallas_call(
        _d1_body,
        grid=(GSTEPS,),
        in_specs=[
            pl.BlockSpec((2, 1, BR, 128), lambda j: (0, j, 0, 0)),
            _row_spec(), _row_spec(), _row_spec(),
            pl.BlockSpec((128, 128), lambda j: (0, 0)),
            pl.BlockSpec((8, 128), lambda j: (0, 0)),
        ],
        out_specs=[_row_spec(), pl.BlockSpec((2, 128), lambda j: (0, 0))],
        out_shape=[jax.ShapeDtypeStruct((GSTEPS, BR, 128), _f32),
                   jax.ShapeDtypeStruct((2, 128), _f32)],
        scratch_shapes=[pltpu.VMEM((2, 128), _f32)],
    )(ysp_t, x_t, rdeg_t, inv_t, wg_t, bg_t)


def _bn_affine(st, g_ref, b_ref, t_ref, tt_ref):
    s16 = jnp.dot(st, tt_ref[...], preferred_element_type=_f32)
    mean = s16[0:1] / N
    var = s16[1:2] / N - mean * mean
    istd = lax.rsqrt(var + 1e-5)
    scale = istd * g_ref[...]
    shift = b_ref[...] - mean * scale
    scale_t = jnp.dot(scale, t_ref[...], preferred_element_type=_f32)
    shift_t = jnp.dot(shift, t_ref[...], preferred_element_type=_f32)
    return scale_t, shift_t


def _d2_body(x1_ref, st_ref, g_ref, b_ref, t_ref, tt_ref,
             w1_ref, b1_ref, w2_ref, x2_ref, st2_ref, acc):
    j = pl.program_id(0)
    scale_t, shift_t = _bn_affine(st_ref[...], g_ref, b_ref, t_ref, tt_ref)
    x1n = x1_ref[0] * scale_t + shift_t
    h = jnp.maximum(
        jnp.dot(x1n, w1_ref[...], preferred_element_type=_f32) + b1_ref[0:1],
        0.0)
    x2 = jnp.dot(h, w2_ref[...], preferred_element_type=_f32) + x1n
    x2_ref[0] = x2

    @pl.when(j == 0)
    def _():
        acc[...] = jnp.zeros_like(acc)

    ssum = jnp.sum(x2, axis=0, keepdims=True)
    ssq = jnp.sum(x2 * x2, axis=0, keepdims=True)
    acc[...] += jnp.concatenate([ssum, ssq], axis=0)

    @pl.when(j == GSTEPS - 1)
    def _():
        st2_ref[...] = acc[...]


def _d2_call(x1_t, st1, g16, b16, t16, tt16, w1_t, b1_t, w2_t):
    return pl.pallas_call(
        _d2_body,
        grid=(GSTEPS,),
        in_specs=[
            _row_spec(),
            pl.BlockSpec((2, 128), lambda j: (0, 0)),
            pl.BlockSpec((1, 16), lambda j: (0, 0)),
            pl.BlockSpec((1, 16), lambda j: (0, 0)),
            pl.BlockSpec((16, 128), lambda j: (0, 0)),
            pl.BlockSpec((128, 16), lambda j: (0, 0)),
            pl.BlockSpec((128, 512), lambda j: (0, 0)),
            pl.BlockSpec((8, 512), lambda j: (0, 0)),
            pl.BlockSpec((512, 128), lambda j: (0, 0)),
        ],
        out_specs=[_row_spec(), pl.BlockSpec((2, 128), lambda j: (0, 0))],
        out_shape=[jax.ShapeDtypeStruct((GSTEPS, BR, 128), _f32),
                   jax.ShapeDtypeStruct((2, 128), _f32)],
        scratch_shapes=[pltpu.VMEM((2, 128), _f32)],
    )(x1_t, st1, g16, b16, t16, tt16, w1_t, b1_t, w2_t)


def _d3a_body(x2_ref, st_ref, g_ref, b_ref, t_ref, tt_ref, rdeg_ref,
              xo_ref, xso_ref):
    scale_t, shift_t = _bn_affine(st_ref[...], g_ref, b_ref, t_ref, tt_ref)
    xn = x2_ref[0] * scale_t + shift_t
    xo_ref[0] = xn
    xso_ref[0] = xn * rdeg_ref[0]


def _d3a_call(x2_t, st2, g16, b16, t16, tt16, rdeg_t):
    return pl.pallas_call(
        _d3a_body,
        grid=(GSTEPS,),
        in_specs=[
            _row_spec(),
            pl.BlockSpec((2, 128), lambda j: (0, 0)),
            pl.BlockSpec((1, 16), lambda j: (0, 0)),
            pl.BlockSpec((1, 16), lambda j: (0, 0)),
            pl.BlockSpec((16, 128), lambda j: (0, 0)),
            pl.BlockSpec((128, 16), lambda j: (0, 0)),
            _row_spec(),
        ],
        out_specs=[_row_spec(), _row_spec()],
        out_shape=[jax.ShapeDtypeStruct((GSTEPS, BR, 128), _f32)] * 2,
    )(x2_t, st2, g16, b16, t16, tt16, rdeg_t)


def _d3b_body(x2_ref, st_ref, g_ref, b_ref, t_ref, tt_ref, wc_ref, bc_ref,
              y_ref):
    scale_t, shift_t = _bn_affine(st_ref[...], g_ref, b_ref, t_ref, tt_ref)
    xn = x2_ref[0] * scale_t + shift_t
    y_ref[0] = (jnp.dot(xn, wc_ref[...], preferred_element_type=_f32)
                + bc_ref[0:1])


def _d3b_call(x2_t, st2, g16, b16, t16, tt16, wc_t, bc_t):
    return pl.pallas_call(
        _d3b_body,
        grid=(GSTEPS,),
        in_specs=[
            _row_spec(),
            pl.BlockSpec((2, 128), lambda j: (0, 0)),
            pl.BlockSpec((1, 16), lambda j: (0, 0)),
            pl.BlockSpec((1, 16), lambda j: (0, 0)),
            pl.BlockSpec((16, 128), lambda j: (0, 0)),
            pl.BlockSpec((128, 16), lambda j: (0, 0)),
            pl.BlockSpec((128, 320), lambda j: (0, 0)),
            pl.BlockSpec((8, 320), lambda j: (0, 0)),
        ],
        out_specs=pl.BlockSpec((1, BR, 320), lambda j: (j, 0, 0)),
        out_shape=jax.ShapeDtypeStruct((GSTEPS, BR, 320), _f32),
    )(x2_t, st2, g16, b16, t16, tt16, wc_t, bc_t)


def _kron8(w):
    return jnp.kron(jnp.eye(8, dtype=_f32), w)


def _tile_bias(b, reps, rows=8):
    return jnp.broadcast_to(jnp.tile(b, reps)[None, :], (rows, b.shape[0] * reps))


def kernel(edge_index, nodes, Wg, bg, bn1_g, bn1_b, W1, b1, W2, bn2_g, bn2_b,
           cls_W, cls_b):
    er = edge_index.reshape(2, NW, ROWS, SUB)

    r8 = jnp.repeat(jnp.eye(8, dtype=_f32), 16, axis=1)
    t16 = jnp.tile(jnp.eye(16, dtype=_f32), (1, 8))
    tt16 = t16.T

    degp = _deg_call(er)
    degp_t = degp.reshape(2, GSTEPS, BR, 8)
    x_t = nodes.reshape(GSTEPS, BR, 128)
    xs_t, rdeg_t, inv_t = _prep_call(degp_t, x_t, r8)

    y_t = None
    for i in range(2):
        xs = xs_t.reshape(N, EMB)
        ysp = _agg_call(xs, er)
        ysp_t = ysp.reshape(2, GSTEPS, BR, 128)
        x1_t, st1 = _d1_call(ysp_t, x_t, rdeg_t, inv_t,
                             _kron8(Wg[i]), _tile_bias(bg[i], 8))
        x2_t, st2 = _d2_call(x1_t, st1,
                             bn1_g[i][None, :], bn1_b[i][None, :], t16, tt16,
                             _kron8(W1[i]), _tile_bias(b1[i], 8),
                             _kron8(W2[i]))
        if i == 0:
            x_t, xs_t = _d3a_call(x2_t, st2, bn2_g[i][None, :],
                                  bn2_b[i][None, :], t16, tt16, rdeg_t)
        else:
            y_t = _d3b_call(x2_t, st2, bn2_g[i][None, :], bn2_b[i][None, :],
                            t16, tt16, _kron8(cls_W),
                            _tile_bias(cls_b, 8))
    return y_t.reshape(N, NUMCLS)
